# baseline (speedup 1.0000x reference)
import functools

import jax
import jax.numpy as jnp
from jax import lax
from jax.experimental import pallas as pl
from jax.experimental.pallas import tpu as pltpu
from jax.experimental.pallas import tpu_sc as plsc

N, E, D, R, L = 10000, 320000, 128, 10, 3

def _take16(v, idx):
    dnums = lax.GatherDimensionNumbers(
        offset_dims=(), collapsed_slice_dims=(0,), start_index_map=(0,))
    return lax.gather(v, idx[:, None], dimension_numbers=dnums,
                      slice_sizes=(1,),
                      mode=lax.GatherScatterMode.PROMISE_IN_BOUNDS)


def _prefix16(v):
    iota = lax.iota(jnp.int32, 16)
    for k in (1, 2, 4, 8):
        shifted = _take16(v, jnp.maximum(iota - k, 0))
        v = v + jnp.where(iota >= k, shifted, 0)
    return v


def _splat16(v, lane):
    return _take16(v, jnp.full((16,), lane, dtype=jnp.int32))


def _splat_to_scalar(v, nbits):
    s = jnp.int32(0)
    for b in range(nbits):
        bit = jnp.any(((v >> b) & 1) == 1).astype(jnp.int32)
        s = s + (bit << b)
    return s

NT = 32
EC = E // NT
K = 128
BA = 80
TRASH = EC + K
CAP = EC + K + 16
NPAD = 10240
DUMMY = N
R16 = 16
NR = NPAD * R16
CW = 8


def _hist_kernel(dst_hbm, et_hbm, ones_hbm, zeros_hbm, hist_hbm,
                 stage_d, stage_e, idx_ref, ones_v, zrows_v, cnt_s):
    cid = lax.axis_index("c")
    sid = lax.axis_index("s")
    wid = sid * 2 + cid
    base = wid * EC
    zr = NR // 16

    pltpu.sync_copy(zeros_hbm, zrows_v)
    pltpu.sync_copy(zrows_v, cnt_s.at[pl.ds(sid * zr, zr)])
    pltpu.sync_copy(ones_hbm, ones_v)
    plsc.subcore_barrier()

    for seg in range(5):
        off = base + seg * 2000
        pltpu.sync_copy(dst_hbm.at[pl.ds(off, 2000)], stage_d)
        pltpu.sync_copy(et_hbm.at[pl.ds(off, 2000)], stage_e)

        def body(i, carry):
            for g in range(BA // 16):
                sl = pl.ds(i * BA + g * 16, 16)
                idx = stage_d[sl] * R16 + stage_e[sl]
                idx_ref[pl.ds(g * 16, 16)] = idx
            pltpu.sync_copy(ones_v, cnt_s.at[idx_ref], add=True)
            return carry

        lax.fori_loop(0, 2000 // BA, body, 0)
    plsc.subcore_barrier()

    pltpu.sync_copy(cnt_s.at[pl.ds(sid * zr, zr)], zrows_v)
    pltpu.sync_copy(zrows_v, hist_hbm.at[cid, pl.ds(sid * zr, zr)])


def _compress_kernel(src_hbm, dst_hbm, et_hbm, dummy_hbm, srclist, dstlist,
                     cnts, src_v, dst_v, et_v, dummy_v, pos_ref, pad_ref,
                     cnt_ref):
    cid = lax.axis_index("c")
    sid = lax.axis_index("s")
    wid = sid * 2 + cid
    base = wid * EC
    iota = lax.iota(jnp.int32, 16)

    pltpu.sync_copy(src_hbm.at[pl.ds(base, EC)], src_v)
    pltpu.sync_copy(dst_hbm.at[pl.ds(base, EC)], dst_v)
    pltpu.sync_copy(et_hbm.at[pl.ds(base, EC)], et_v)
    pltpu.sync_copy(dummy_hbm, dummy_v)

    cntv = jnp.zeros((16,), dtype=jnp.int32)
    for r in range(R):
        fbase = (r * NT + wid) * CAP

        def body(b, off):
            for g in range(BA // 16):
                sl = pl.ds(b * BA + g * 16, 16)
                m = et_v[sl] == r
                pos = _prefix16(jnp.where(m, 1, 0))
                idx = jnp.where(m, off + pos - 1, TRASH + iota)
                pos_ref[pl.ds(g * 16, 16)] = fbase + idx
                off = off + _splat16(pos, 15)
            sl = pl.ds(b * BA, BA)
            pltpu.sync_copy(src_v.at[sl], srclist.at[pos_ref])
            pltpu.sync_copy(dst_v.at[sl], dstlist.at[pos_ref])
            return off

        off = lax.fori_loop(0, EC // BA, body, jnp.zeros((16,), jnp.int32))
        target = (off + (K - 1)) & (-K)

        for i in range((K // 16) + 1):
            idx = off + i * 16 + iota
            idx = jnp.where(idx < target, idx, TRASH + iota)
            pad_ref[...] = fbase + idx
            pltpu.sync_copy(dummy_v.at[pl.ds(0, 16)], srclist.at[pad_ref])
            pltpu.sync_copy(dummy_v.at[pl.ds(16, 16)], dstlist.at[pad_ref])
        cntv = jnp.where(iota == r, target, cntv)
    cnt_ref[...] = cntv
    pltpu.sync_copy(cnt_ref, cnts.at[wid])


def _segsum_kernel(x_hbm, srclist, dstlist, cnts, zeros_hbm, s_hbm,
                   sidx_v, didx_v, rows_v, zero_v, cnt_ref, sem, acc_s):
    cid = lax.axis_index("c")
    sid = lax.axis_index("s")
    iota = lax.iota(jnp.int32, 16)
    row0 = sid * (NPAD // 16)

    pltpu.sync_copy(zeros_hbm, zero_v)
    for p in range(R // 2):
        r = 2 * p + cid
        for z in range(NPAD // 16 // K):
            pltpu.sync_copy(zero_v, acc_s.at[pl.ds(row0 + z * K, K)])
        plsc.subcore_barrier()

        for cc in range(2):
            c = 2 * sid + cc
            pltpu.sync_copy(cnts.at[c], cnt_ref)
            cv = cnt_ref[...]
            n_it = lax.select(cid == 0, cv[2 * p], cv[2 * p + 1]) >> 7

            fbase = (r * NT + c) * CAP

            def body(i, carry):
                off = fbase + i * K
                pltpu.sync_copy(srclist.at[pl.ds(off, K)], sidx_v)
                pltpu.sync_copy(dstlist.at[pl.ds(off, K)], didx_v)
                pltpu.async_copy(x_hbm.at[sidx_v], rows_v, sem).wait()
                pltpu.sync_copy(rows_v, acc_s.at[didx_v], add=True)
                return carry

            lax.fori_loop(0, n_it, body, 0)
        plsc.subcore_barrier()

        for z in range(NPAD // 16 // K):
            sl = pl.ds(row0 + z * K, K)
            pltpu.sync_copy(acc_s.at[sl], rows_v)
            pltpu.sync_copy(rows_v, s_hbm.at[r, sl])
        plsc.subcore_barrier()


def _tc_layer_kernel(x_ref, s_ref, hist_ref, wrel_ref, wroot_ref, b_ref,
                     g_ref, bt_ref, out_ref):
    x = x_ref[...]
    h3 = hist_ref[...]
    cnt = h3[0] + h3[1]
    recip = 1.0 / jnp.maximum(cnt, 1.0)
    acc = jnp.dot(x, wroot_ref[...], preferred_element_type=jnp.float32)
    acc += b_ref[...]
    for r in range(R):
        m = s_ref[r] * recip[:, r:r + 1]
        acc += jnp.dot(m, wrel_ref[r], preferred_element_type=jnp.float32)
    h = acc + x
    mu = jnp.mean(h, axis=-1, keepdims=True)
    var = jnp.mean((h - mu) ** 2, axis=-1, keepdims=True)
    y = (h - mu) * lax.rsqrt(var + 1e-5) * g_ref[...] + bt_ref[...]
    out_ref[...] = jnp.maximum(y, 0.0)


def _make_sc_calls():
    mesh = plsc.VectorSubcoreMesh(core_axis_name="c", subcore_axis_name="s")
    f32, i32 = jnp.float32, jnp.int32
    hist = pl.kernel(
        _hist_kernel, mesh=mesh,
        out_type=jax.ShapeDtypeStruct((2, NR), f32),
        scratch_types=[
            pltpu.VMEM((2000,), i32),
            pltpu.VMEM((2000,), i32),
            pltpu.VMEM((BA,), i32),
            pltpu.VMEM((BA,), f32),
            pltpu.VMEM((NR // 16,), f32),
            pltpu.VMEM_SHARED((NR,), f32),
        ],
    )
    compress = pl.kernel(
        _compress_kernel, mesh=mesh,
        out_type=(
            jax.ShapeDtypeStruct((R * NT * CAP,), i32),
            jax.ShapeDtypeStruct((R * NT * CAP,), i32),
            jax.ShapeDtypeStruct((NT, 16), i32),
        ),
        scratch_types=[
            pltpu.VMEM((EC,), i32),
            pltpu.VMEM((EC,), i32),
            pltpu.VMEM((EC,), i32),
            pltpu.VMEM((32,), i32),
            pltpu.VMEM((BA,), i32),
            pltpu.VMEM((16,), i32),
            pltpu.VMEM((16,), i32),
        ],
    )
    segsum = pl.kernel(
        _segsum_kernel, mesh=mesh,
        out_type=jax.ShapeDtypeStruct((R, NPAD, D), f32),
        scratch_types=[
            pltpu.VMEM((K,), i32),
            pltpu.VMEM((K,), i32),
            pltpu.VMEM((K, D), f32),
            pltpu.VMEM((K, D), f32),
            pltpu.VMEM((16,), i32),
            pltpu.SemaphoreType.DMA,
            pltpu.VMEM_SHARED((NPAD, D), f32),
        ],
    )
    return hist, compress, segsum


def kernel(node_features, edge_index, edge_type, batch_indices,
           Wrel_0, Wroot_0, b_0, ln_g_0, ln_b_0,
           Wrel_1, Wroot_1, b_1, ln_g_1, ln_b_1,
           Wrel_2, Wroot_2, b_2, ln_g_2, ln_b_2):
    src = edge_index[0].astype(jnp.int32)
    dst = edge_index[1].astype(jnp.int32)
    et = edge_type.astype(jnp.int32)
    hist_call, compress_call, segsum_call = _make_sc_calls()

    ones_a = jnp.ones((BA,), dtype=jnp.float32)
    zeros_a = jnp.zeros((NR // 16,), dtype=jnp.float32)
    dummy_a = jnp.concatenate([jnp.zeros((16,), jnp.int32),
                               jnp.full((16,), DUMMY, jnp.int32)])
    zeros_k = jnp.zeros((K, D), dtype=jnp.float32)

    hist = hist_call(dst, et, ones_a, zeros_a)
    srclist, dstlist, cnts = compress_call(src, dst, et, dummy_a)
    hist3 = hist.reshape(2, NPAD, R16)

    x = jnp.pad(node_features, ((0, NPAD - N), (0, 0)))
    params = [
        (Wrel_0, Wroot_0, b_0, ln_g_0, ln_b_0),
        (Wrel_1, Wroot_1, b_1, ln_g_1, ln_b_1),
        (Wrel_2, Wroot_2, b_2, ln_g_2, ln_b_2),
    ]

    BN = 512
    grid = (NPAD // BN,)
    tc_call = pl.pallas_call(
        _tc_layer_kernel,
        grid=grid,
        in_specs=[
            pl.BlockSpec((BN, D), lambda i: (i, 0)),
            pl.BlockSpec((R, BN, D), lambda i: (0, i, 0)),
            pl.BlockSpec((2, BN, R16), lambda i: (0, i, 0)),
            pl.BlockSpec((R, D, D), lambda i: (0, 0, 0)),
            pl.BlockSpec((D, D), lambda i: (0, 0)),
            pl.BlockSpec((1, D), lambda i: (0, 0)),
            pl.BlockSpec((1, D), lambda i: (0, 0)),
            pl.BlockSpec((1, D), lambda i: (0, 0)),
        ],
        out_specs=pl.BlockSpec((BN, D), lambda i: (i, 0)),
        out_shape=jax.ShapeDtypeStruct((NPAD, D), jnp.float32),
    )

    for (Wrel, Wroot, b, g, bt) in params:
        s = segsum_call(x, srclist, dstlist, cnts, zeros_k)
        x = tc_call(x, s, hist3, Wrel, Wroot, b.reshape(1, D),
                    g.reshape(1, D), bt.reshape(1, D))
    return x[:N]


# speedup vs baseline: 1.0017x; 1.0017x over previous
import functools

import jax
import jax.numpy as jnp
from jax import lax
from jax.experimental import pallas as pl
from jax.experimental.pallas import tpu as pltpu
from jax.experimental.pallas import tpu_sc as plsc

N, E, D, R, L = 10000, 320000, 128, 10, 3

def _take16(v, idx):
    dnums = lax.GatherDimensionNumbers(
        offset_dims=(), collapsed_slice_dims=(0,), start_index_map=(0,))
    return lax.gather(v, idx[:, None], dimension_numbers=dnums,
                      slice_sizes=(1,),
                      mode=lax.GatherScatterMode.PROMISE_IN_BOUNDS)


def _prefix16(v):
    iota = lax.iota(jnp.int32, 16)
    for k in (1, 2, 4, 8):
        shifted = _take16(v, jnp.maximum(iota - k, 0))
        v = v + jnp.where(iota >= k, shifted, 0)
    return v


def _splat16(v, lane):
    return _take16(v, jnp.full((16,), lane, dtype=jnp.int32))


def _splat_to_scalar(v, nbits):
    s = jnp.int32(0)
    for b in range(nbits):
        bit = jnp.any(((v >> b) & 1) == 1).astype(jnp.int32)
        s = s + (bit << b)
    return s

NT = 32
EC = E // NT
K = 128
BA = 80
TRASH = EC + K
CAP = EC + K + 16
NPAD = 10240
DUMMY = N
R16 = 16
NR = NPAD * R16
CW = 8


def _hist_kernel(dst_hbm, et_hbm, ones_hbm, zeros_hbm, hist_hbm,
                 stage_d, stage_e, idx_ref, ones_v, zrows_v, cnt_s):
    cid = lax.axis_index("c")
    sid = lax.axis_index("s")
    wid = sid * 2 + cid
    base = wid * EC
    zr = NR // 16

    pltpu.sync_copy(zeros_hbm, zrows_v)
    pltpu.sync_copy(zrows_v, cnt_s.at[pl.ds(sid * zr, zr)])
    pltpu.sync_copy(ones_hbm, ones_v)
    plsc.subcore_barrier()

    for seg in range(5):
        off = base + seg * 2000
        pltpu.sync_copy(dst_hbm.at[pl.ds(off, 2000)], stage_d)
        pltpu.sync_copy(et_hbm.at[pl.ds(off, 2000)], stage_e)

        def body(i, carry):
            for g in range(BA // 16):
                sl = pl.ds(i * BA + g * 16, 16)
                idx = stage_d[sl] * R16 + stage_e[sl]
                idx_ref[pl.ds(g * 16, 16)] = idx
            pltpu.sync_copy(ones_v, cnt_s.at[idx_ref], add=True)
            return carry

        lax.fori_loop(0, 2000 // BA, body, 0)
    plsc.subcore_barrier()

    pltpu.sync_copy(cnt_s.at[pl.ds(sid * zr, zr)], zrows_v)
    pltpu.sync_copy(zrows_v, hist_hbm.at[cid, pl.ds(sid * zr, zr)])


def _compress_kernel(src_hbm, dst_hbm, et_hbm, dummy_hbm, srclist, dstlist,
                     cnts, src_v, dst_v, et_v, dummy_v, pos_ref, pad_ref,
                     cnt_ref):
    cid = lax.axis_index("c")
    sid = lax.axis_index("s")
    wid = sid * 2 + cid
    base = wid * EC
    iota = lax.iota(jnp.int32, 16)

    pltpu.sync_copy(src_hbm.at[pl.ds(base, EC)], src_v)
    pltpu.sync_copy(dst_hbm.at[pl.ds(base, EC)], dst_v)
    pltpu.sync_copy(et_hbm.at[pl.ds(base, EC)], et_v)
    pltpu.sync_copy(dummy_hbm, dummy_v)

    def cbody(i, carry):
        et16 = et_v[pl.ds(i * 16, 16)]
        return tuple(c + jnp.where(et16 == r, 1, 0)
                     for r, c in enumerate(carry))

    counts = lax.fori_loop(0, EC // 16, cbody,
                           tuple(jnp.zeros((16,), jnp.int32)
                                 for _ in range(R)))

    cntv = jnp.zeros((16,), dtype=jnp.int32)
    for r in range(R):
        fbase = (r * NT + wid) * CAP
        pr = _prefix16(counts[r])
        base_v = pr - counts[r]
        total = _splat16(pr, 15)
        target = (total + (K - 1)) & (-K)

        def body(b, run):
            for g in range(BA // 16):
                sl = pl.ds(b * BA + g * 16, 16)
                m = et_v[sl] == r
                idx = jnp.where(m, base_v + run, TRASH + iota)
                pos_ref[pl.ds(g * 16, 16)] = fbase + idx
                run = run + jnp.where(m, 1, 0)
            sl = pl.ds(b * BA, BA)
            pltpu.sync_copy(src_v.at[sl], srclist.at[pos_ref])
            pltpu.sync_copy(dst_v.at[sl], dstlist.at[pos_ref])
            return run

        lax.fori_loop(0, EC // BA, body, jnp.zeros((16,), jnp.int32))

        for i in range((K // 16) + 1):
            idx = total + i * 16 + iota
            idx = jnp.where(idx < target, idx, TRASH + iota)
            pad_ref[...] = fbase + idx
            pltpu.sync_copy(dummy_v.at[pl.ds(0, 16)], srclist.at[pad_ref])
            pltpu.sync_copy(dummy_v.at[pl.ds(16, 16)], dstlist.at[pad_ref])
        cntv = jnp.where(iota == r, target, cntv)
    cnt_ref[...] = cntv
    pltpu.sync_copy(cnt_ref, cnts.at[wid])


def _segsum_kernel(x_hbm, srclist, dstlist, cnts, zeros_hbm, s_hbm,
                   sidx_v, didx_v, rows_v, zero_v, cnt_ref, sem, acc_s):
    cid = lax.axis_index("c")
    sid = lax.axis_index("s")
    iota = lax.iota(jnp.int32, 16)
    row0 = sid * (NPAD // 16)

    pltpu.sync_copy(zeros_hbm, zero_v)
    for p in range(R // 2):
        r = 2 * p + cid
        for z in range(NPAD // 16 // K):
            pltpu.sync_copy(zero_v, acc_s.at[pl.ds(row0 + z * K, K)])
        plsc.subcore_barrier()

        for cc in range(2):
            c = 2 * sid + cc
            pltpu.sync_copy(cnts.at[c], cnt_ref)
            cv = cnt_ref[...]
            n_it = lax.select(cid == 0, cv[2 * p], cv[2 * p + 1]) >> 7

            fbase = (r * NT + c) * CAP

            def body(i, carry):
                off = fbase + i * K
                pltpu.sync_copy(srclist.at[pl.ds(off, K)], sidx_v)
                pltpu.sync_copy(dstlist.at[pl.ds(off, K)], didx_v)
                pltpu.async_copy(x_hbm.at[sidx_v], rows_v, sem).wait()
                pltpu.sync_copy(rows_v, acc_s.at[didx_v], add=True)
                return carry

            lax.fori_loop(0, n_it, body, 0)
        plsc.subcore_barrier()

        for z in range(NPAD // 16 // K):
            sl = pl.ds(row0 + z * K, K)
            pltpu.sync_copy(acc_s.at[sl], rows_v)
            pltpu.sync_copy(rows_v, s_hbm.at[r, sl])
        plsc.subcore_barrier()


def _tc_layer_kernel(x_ref, s_ref, hist_ref, wrel_ref, wroot_ref, b_ref,
                     g_ref, bt_ref, out_ref):
    x = x_ref[...]
    h3 = hist_ref[...]
    cnt = h3[0] + h3[1]
    recip = 1.0 / jnp.maximum(cnt, 1.0)
    acc = jnp.dot(x, wroot_ref[...], preferred_element_type=jnp.float32)
    acc += b_ref[...]
    for r in range(R):
        m = s_ref[r] * recip[:, r:r + 1]
        acc += jnp.dot(m, wrel_ref[r], preferred_element_type=jnp.float32)
    h = acc + x
    mu = jnp.mean(h, axis=-1, keepdims=True)
    var = jnp.mean((h - mu) ** 2, axis=-1, keepdims=True)
    y = (h - mu) * lax.rsqrt(var + 1e-5) * g_ref[...] + bt_ref[...]
    out_ref[...] = jnp.maximum(y, 0.0)


def _make_sc_calls():
    mesh = plsc.VectorSubcoreMesh(core_axis_name="c", subcore_axis_name="s")
    f32, i32 = jnp.float32, jnp.int32
    hist = pl.kernel(
        _hist_kernel, mesh=mesh,
        out_type=jax.ShapeDtypeStruct((2, NR), f32),
        scratch_types=[
            pltpu.VMEM((2000,), i32),
            pltpu.VMEM((2000,), i32),
            pltpu.VMEM((BA,), i32),
            pltpu.VMEM((BA,), f32),
            pltpu.VMEM((NR // 16,), f32),
            pltpu.VMEM_SHARED((NR,), f32),
        ],
    )
    compress = pl.kernel(
        _compress_kernel, mesh=mesh,
        out_type=(
            jax.ShapeDtypeStruct((R * NT * CAP,), i32),
            jax.ShapeDtypeStruct((R * NT * CAP,), i32),
            jax.ShapeDtypeStruct((NT, 16), i32),
        ),
        scratch_types=[
            pltpu.VMEM((EC,), i32),
            pltpu.VMEM((EC,), i32),
            pltpu.VMEM((EC,), i32),
            pltpu.VMEM((32,), i32),
            pltpu.VMEM((BA,), i32),
            pltpu.VMEM((16,), i32),
            pltpu.VMEM((16,), i32),
        ],
    )
    segsum = pl.kernel(
        _segsum_kernel, mesh=mesh,
        out_type=jax.ShapeDtypeStruct((R, NPAD, D), f32),
        scratch_types=[
            pltpu.VMEM((K,), i32),
            pltpu.VMEM((K,), i32),
            pltpu.VMEM((K, D), f32),
            pltpu.VMEM((K, D), f32),
            pltpu.VMEM((16,), i32),
            pltpu.SemaphoreType.DMA,
            pltpu.VMEM_SHARED((NPAD, D), f32),
        ],
    )
    return hist, compress, segsum


def kernel(node_features, edge_index, edge_type, batch_indices,
           Wrel_0, Wroot_0, b_0, ln_g_0, ln_b_0,
           Wrel_1, Wroot_1, b_1, ln_g_1, ln_b_1,
           Wrel_2, Wroot_2, b_2, ln_g_2, ln_b_2):
    src = edge_index[0].astype(jnp.int32)
    dst = edge_index[1].astype(jnp.int32)
    et = edge_type.astype(jnp.int32)
    hist_call, compress_call, segsum_call = _make_sc_calls()

    ones_a = jnp.ones((BA,), dtype=jnp.float32)
    zeros_a = jnp.zeros((NR // 16,), dtype=jnp.float32)
    dummy_a = jnp.concatenate([jnp.zeros((16,), jnp.int32),
                               jnp.full((16,), DUMMY, jnp.int32)])
    zeros_k = jnp.zeros((K, D), dtype=jnp.float32)

    hist = hist_call(dst, et, ones_a, zeros_a)
    srclist, dstlist, cnts = compress_call(src, dst, et, dummy_a)
    hist3 = hist.reshape(2, NPAD, R16)

    x = jnp.pad(node_features, ((0, NPAD - N), (0, 0)))
    params = [
        (Wrel_0, Wroot_0, b_0, ln_g_0, ln_b_0),
        (Wrel_1, Wroot_1, b_1, ln_g_1, ln_b_1),
        (Wrel_2, Wroot_2, b_2, ln_g_2, ln_b_2),
    ]

    BN = 512
    grid = (NPAD // BN,)
    tc_call = pl.pallas_call(
        _tc_layer_kernel,
        grid=grid,
        in_specs=[
            pl.BlockSpec((BN, D), lambda i: (i, 0)),
            pl.BlockSpec((R, BN, D), lambda i: (0, i, 0)),
            pl.BlockSpec((2, BN, R16), lambda i: (0, i, 0)),
            pl.BlockSpec((R, D, D), lambda i: (0, 0, 0)),
            pl.BlockSpec((D, D), lambda i: (0, 0)),
            pl.BlockSpec((1, D), lambda i: (0, 0)),
            pl.BlockSpec((1, D), lambda i: (0, 0)),
            pl.BlockSpec((1, D), lambda i: (0, 0)),
        ],
        out_specs=pl.BlockSpec((BN, D), lambda i: (i, 0)),
        out_shape=jax.ShapeDtypeStruct((NPAD, D), jnp.float32),
    )

    for (Wrel, Wroot, b, g, bt) in params:
        s = segsum_call(x, srclist, dstlist, cnts, zeros_k)
        x = tc_call(x, s, hist3, Wrel, Wroot, b.reshape(1, D),
                    g.reshape(1, D), bt.reshape(1, D))
    return x[:N]


# speedup vs baseline: 12.2016x; 12.1813x over previous
import functools

import jax
import jax.numpy as jnp
from jax import lax
from jax.experimental import pallas as pl
from jax.experimental.pallas import tpu as pltpu
from jax.experimental.pallas import tpu_sc as plsc

N, E, D, R, L = 10000, 320000, 128, 10, 3

def _take16(v, idx):
    dnums = lax.GatherDimensionNumbers(
        offset_dims=(), collapsed_slice_dims=(0,), start_index_map=(0,))
    return lax.gather(v, idx[:, None], dimension_numbers=dnums,
                      slice_sizes=(1,),
                      mode=lax.GatherScatterMode.PROMISE_IN_BOUNDS)


def _prefix16(v):
    iota = lax.iota(jnp.int32, 16)
    for k in (1, 2, 4, 8):
        shifted = _take16(v, jnp.maximum(iota - k, 0))
        v = v + jnp.where(iota >= k, shifted, 0)
    return v


def _splat16(v, lane):
    return _take16(v, jnp.full((16,), lane, dtype=jnp.int32))


def _splat_to_scalar(v, nbits):
    s = jnp.int32(0)
    for b in range(nbits):
        bit = jnp.any(((v >> b) & 1) == 1).astype(jnp.int32)
        s = s + (bit << b)
    return s

NT = 32
EC = E // NT
K = 128
BA = 80
TRASH = EC + K
CAP = EC + K + 16
NPAD = 10240
DUMMY = N
R16 = 16
NR = NPAD * R16
CW = 8


def _hist_kernel(dst_hbm, et_hbm, ones_hbm, zeros_hbm, hist_hbm,
                 stage_d, stage_e, idx_ref, ones_v, zrows_v, cnt_s):
    cid = lax.axis_index("c")
    sid = lax.axis_index("s")
    wid = sid * 2 + cid
    base = wid * EC
    zr = NR // 16

    pltpu.sync_copy(zeros_hbm, zrows_v)
    pltpu.sync_copy(zrows_v, cnt_s.at[pl.ds(sid * zr, zr)])
    pltpu.sync_copy(ones_hbm, ones_v)
    plsc.subcore_barrier()

    for seg in range(5):
        off = base + seg * 2000
        pltpu.sync_copy(dst_hbm.at[pl.ds(off, 2000)], stage_d)
        pltpu.sync_copy(et_hbm.at[pl.ds(off, 2000)], stage_e)

        def body(i, carry):
            for g in range(BA // 16):
                sl = pl.ds(i * BA + g * 16, 16)
                idx = stage_d[sl] * R16 + stage_e[sl]
                idx_ref[pl.ds(g * 16, 16)] = idx
            pltpu.sync_copy(ones_v, cnt_s.at[idx_ref], add=True)
            return carry

        lax.fori_loop(0, 2000 // BA, body, 0)
    plsc.subcore_barrier()

    pltpu.sync_copy(cnt_s.at[pl.ds(sid * zr, zr)], zrows_v)
    pltpu.sync_copy(zrows_v, hist_hbm.at[cid, pl.ds(sid * zr, zr)])


def _compress_kernel(src_hbm, dst_hbm, et_hbm, dummy_hbm, srclist, dstlist,
                     cnts, src_v, dst_v, et_v, dummy_v, pos_ref, pad_ref,
                     cnt_ref, srcbuf, dstbuf, slist_s, dlist_s):
    cid = lax.axis_index("c")
    sid = lax.axis_index("s")
    wid = sid * 2 + cid
    base = wid * EC
    iota = lax.iota(jnp.int32, 16)

    pltpu.sync_copy(src_hbm.at[pl.ds(base, EC)], src_v)
    pltpu.sync_copy(dst_hbm.at[pl.ds(base, EC)], dst_v)
    pltpu.sync_copy(et_hbm.at[pl.ds(base, EC)], et_v)
    pltpu.sync_copy(dummy_hbm, dummy_v)

    def cbody(i, carry):
        et16 = et_v[pl.ds(i * 16, 16)]
        return tuple(c + jnp.where(et16 == r, 1, 0)
                     for r, c in enumerate(carry))

    counts = lax.fori_loop(0, EC // 16, cbody,
                           tuple(jnp.zeros((16,), jnp.int32)
                                 for _ in range(R)))

    cntv = jnp.zeros((16,), dtype=jnp.int32)
    sbase = sid * CAP
    for r in range(R):
        fbase = (r * NT + wid) * CAP
        pr = _prefix16(counts[r])
        base_v = pr - counts[r]
        total = _splat16(pr, 15)
        target = (total + (K - 1)) & (-K)

        def body(b, run):
            for g in range(BA // 16):
                sl = pl.ds(b * BA + g * 16, 16)
                m = et_v[sl] == r
                idx = jnp.where(m, base_v + run, TRASH + iota)
                pos_ref[pl.ds(g * 16, 16)] = sbase + idx
                run = run + jnp.where(m, 1, 0)
            sl = pl.ds(b * BA, BA)
            pltpu.sync_copy(src_v.at[sl], slist_s.at[pos_ref])
            pltpu.sync_copy(dst_v.at[sl], dlist_s.at[pos_ref])
            return run

        lax.fori_loop(0, EC // BA, body, jnp.zeros((16,), jnp.int32))

        for i in range((K // 16) + 1):
            idx = total + i * 16 + iota
            idx = jnp.where(idx < target, idx, TRASH + iota)
            pad_ref[...] = sbase + idx
            pltpu.sync_copy(dummy_v.at[pl.ds(0, 16)], slist_s.at[pad_ref])
            pltpu.sync_copy(dummy_v.at[pl.ds(16, 16)], dlist_s.at[pad_ref])
        cntv = jnp.where(iota == r, target, cntv)
        pltpu.sync_copy(slist_s.at[pl.ds(sbase, CAP)], srcbuf)
        pltpu.sync_copy(srcbuf, srclist.at[pl.ds(fbase, CAP)])
        pltpu.sync_copy(dlist_s.at[pl.ds(sbase, CAP)], dstbuf)
        pltpu.sync_copy(dstbuf, dstlist.at[pl.ds(fbase, CAP)])
    cnt_ref[...] = cntv
    pltpu.sync_copy(cnt_ref, cnts.at[wid])


def _segsum_kernel(x_hbm, srclist, dstlist, cnts, zeros_hbm, s_hbm,
                   sidx_v, didx_v, rows_v, zero_v, cnt_ref, sem, acc_s):
    cid = lax.axis_index("c")
    sid = lax.axis_index("s")
    iota = lax.iota(jnp.int32, 16)
    row0 = sid * (NPAD // 16)

    pltpu.sync_copy(zeros_hbm, zero_v)
    for p in range(R // 2):
        r = 2 * p + cid
        for z in range(NPAD // 16 // K):
            pltpu.sync_copy(zero_v, acc_s.at[pl.ds(row0 + z * K, K)])
        plsc.subcore_barrier()

        for cc in range(2):
            c = 2 * sid + cc
            pltpu.sync_copy(cnts.at[c], cnt_ref)
            cv = cnt_ref[...]
            n_it = lax.select(cid == 0, cv[2 * p], cv[2 * p + 1]) >> 7

            fbase = (r * NT + c) * CAP

            def body(i, carry):
                off = fbase + i * K
                pltpu.sync_copy(srclist.at[pl.ds(off, K)], sidx_v)
                pltpu.sync_copy(dstlist.at[pl.ds(off, K)], didx_v)
                pltpu.async_copy(x_hbm.at[sidx_v], rows_v, sem).wait()
                pltpu.sync_copy(rows_v, acc_s.at[didx_v], add=True)
                return carry

            lax.fori_loop(0, n_it, body, 0)
        plsc.subcore_barrier()

        for z in range(NPAD // 16 // K):
            sl = pl.ds(row0 + z * K, K)
            pltpu.sync_copy(acc_s.at[sl], rows_v)
            pltpu.sync_copy(rows_v, s_hbm.at[r, sl])
        plsc.subcore_barrier()


def _tc_layer_kernel(x_ref, s_ref, hist_ref, wrel_ref, wroot_ref, b_ref,
                     g_ref, bt_ref, out_ref):
    x = x_ref[...]
    h3 = hist_ref[...]
    cnt = h3[0] + h3[1]
    recip = 1.0 / jnp.maximum(cnt, 1.0)
    acc = jnp.dot(x, wroot_ref[...], preferred_element_type=jnp.float32)
    acc += b_ref[...]
    for r in range(R):
        m = s_ref[r] * recip[:, r:r + 1]
        acc += jnp.dot(m, wrel_ref[r], preferred_element_type=jnp.float32)
    h = acc + x
    mu = jnp.mean(h, axis=-1, keepdims=True)
    var = jnp.mean((h - mu) ** 2, axis=-1, keepdims=True)
    y = (h - mu) * lax.rsqrt(var + 1e-5) * g_ref[...] + bt_ref[...]
    out_ref[...] = jnp.maximum(y, 0.0)


def _make_sc_calls():
    mesh = plsc.VectorSubcoreMesh(core_axis_name="c", subcore_axis_name="s")
    f32, i32 = jnp.float32, jnp.int32
    hist = pl.kernel(
        _hist_kernel, mesh=mesh,
        out_type=jax.ShapeDtypeStruct((2, NR), f32),
        scratch_types=[
            pltpu.VMEM((2000,), i32),
            pltpu.VMEM((2000,), i32),
            pltpu.VMEM((BA,), i32),
            pltpu.VMEM((BA,), f32),
            pltpu.VMEM((NR // 16,), f32),
            pltpu.VMEM_SHARED((NR,), f32),
        ],
    )
    compress = pl.kernel(
        _compress_kernel, mesh=mesh,
        out_type=(
            jax.ShapeDtypeStruct((R * NT * CAP,), i32),
            jax.ShapeDtypeStruct((R * NT * CAP,), i32),
            jax.ShapeDtypeStruct((NT, 16), i32),
        ),
        scratch_types=[
            pltpu.VMEM((EC,), i32),
            pltpu.VMEM((EC,), i32),
            pltpu.VMEM((EC,), i32),
            pltpu.VMEM((32,), i32),
            pltpu.VMEM((BA,), i32),
            pltpu.VMEM((16,), i32),
            pltpu.VMEM((16,), i32),
            pltpu.VMEM((CAP,), i32),
            pltpu.VMEM((CAP,), i32),
            pltpu.VMEM_SHARED((16 * CAP,), i32),
            pltpu.VMEM_SHARED((16 * CAP,), i32),
        ],
    )
    segsum = pl.kernel(
        _segsum_kernel, mesh=mesh,
        out_type=jax.ShapeDtypeStruct((R, NPAD, D), f32),
        scratch_types=[
            pltpu.VMEM((K,), i32),
            pltpu.VMEM((K,), i32),
            pltpu.VMEM((K, D), f32),
            pltpu.VMEM((K, D), f32),
            pltpu.VMEM((16,), i32),
            pltpu.SemaphoreType.DMA,
            pltpu.VMEM_SHARED((NPAD, D), f32),
        ],
    )
    return hist, compress, segsum


def kernel(node_features, edge_index, edge_type, batch_indices,
           Wrel_0, Wroot_0, b_0, ln_g_0, ln_b_0,
           Wrel_1, Wroot_1, b_1, ln_g_1, ln_b_1,
           Wrel_2, Wroot_2, b_2, ln_g_2, ln_b_2):
    src = edge_index[0].astype(jnp.int32)
    dst = edge_index[1].astype(jnp.int32)
    et = edge_type.astype(jnp.int32)
    hist_call, compress_call, segsum_call = _make_sc_calls()

    ones_a = jnp.ones((BA,), dtype=jnp.float32)
    zeros_a = jnp.zeros((NR // 16,), dtype=jnp.float32)
    dummy_a = jnp.concatenate([jnp.zeros((16,), jnp.int32),
                               jnp.full((16,), DUMMY, jnp.int32)])
    zeros_k = jnp.zeros((K, D), dtype=jnp.float32)

    hist = hist_call(dst, et, ones_a, zeros_a)
    srclist, dstlist, cnts = compress_call(src, dst, et, dummy_a)
    hist3 = hist.reshape(2, NPAD, R16)

    x = jnp.pad(node_features, ((0, NPAD - N), (0, 0)))
    params = [
        (Wrel_0, Wroot_0, b_0, ln_g_0, ln_b_0),
        (Wrel_1, Wroot_1, b_1, ln_g_1, ln_b_1),
        (Wrel_2, Wroot_2, b_2, ln_g_2, ln_b_2),
    ]

    BN = 512
    grid = (NPAD // BN,)
    tc_call = pl.pallas_call(
        _tc_layer_kernel,
        grid=grid,
        in_specs=[
            pl.BlockSpec((BN, D), lambda i: (i, 0)),
            pl.BlockSpec((R, BN, D), lambda i: (0, i, 0)),
            pl.BlockSpec((2, BN, R16), lambda i: (0, i, 0)),
            pl.BlockSpec((R, D, D), lambda i: (0, 0, 0)),
            pl.BlockSpec((D, D), lambda i: (0, 0)),
            pl.BlockSpec((1, D), lambda i: (0, 0)),
            pl.BlockSpec((1, D), lambda i: (0, 0)),
            pl.BlockSpec((1, D), lambda i: (0, 0)),
        ],
        out_specs=pl.BlockSpec((BN, D), lambda i: (i, 0)),
        out_shape=jax.ShapeDtypeStruct((NPAD, D), jnp.float32),
    )

    for (Wrel, Wroot, b, g, bt) in params:
        s = segsum_call(x, srclist, dstlist, cnts, zeros_k)
        x = tc_call(x, s, hist3, Wrel, Wroot, b.reshape(1, D),
                    g.reshape(1, D), bt.reshape(1, D))
    return x[:N]
